# baseline (speedup 1.0000x reference)
import functools

import jax
import jax.numpy as jnp
from jax import lax
from jax.experimental import pallas as pl
from jax.experimental.pallas import tpu as pltpu
from jax.experimental.pallas import tpu_sc as plsc

N = 10000
E = 160000
D = 256
DIM1 = 512
R = 3
NCLS = 10
EPS = 0.1

NPAD = 10240
W = 128
NSUB = 16
NCORE = 2
BE = 128
NB = 80
PE = NSUB * NB * BE
ROWS_PT = NPAD // NSUB
BN = 1024
GRID = NPAD // BN


def _make_segsum(nb):
    nbh = 40
    nstage = nb // nbh
    mesh = plsc.VectorSubcoreMesh(core_axis_name="c", subcore_axis_name="s")

    @functools.partial(
        pl.kernel,
        mesh=mesh,
        out_type=jax.ShapeDtypeStruct((NCORE, NPAD, W), jnp.float32),
        scratch_types=[
            pltpu.VMEM((nbh, BE), jnp.int32),
            pltpu.VMEM((nbh, BE), jnp.int32),
            pltpu.VMEM((BE, W), jnp.float32),
            pltpu.VMEM((BE, W), jnp.float32),
            pltpu.VMEM_SHARED((NPAD, W), jnp.float32),
            pltpu.SemaphoreType.DMA,
            pltpu.SemaphoreType.DMA,
            pltpu.SemaphoreType.DMA,
            pltpu.SemaphoreType.DMA,
        ],
    )
    def segsum(y4, srci, dsti, zeros, out, srcv, dstv, rb0, rb1,
               acc, gs0, gs1, ss0, ss1):
        c = lax.axis_index("c")
        s = lax.axis_index("s")
        r0 = s * ROWS_PT
        pltpu.sync_copy(zeros.at[pl.ds(r0, ROWS_PT)], acc.at[pl.ds(r0, ROWS_PT)])
        plsc.subcore_barrier()
        for half in range(nstage):
            h0 = half * nbh
            pltpu.sync_copy(srci.at[c, s, pl.ds(h0, nbh)], srcv)
            pltpu.sync_copy(dsti.at[c, s, pl.ds(h0, nbh)], dstv)
            pltpu.async_copy(y4.at[srcv.at[0]], rb0, gs0)
            pltpu.async_copy(y4.at[srcv.at[1]], rb1, gs1)

            def batch2(k, carry):
                b = 2 * k
                nxt0 = jnp.where(b + 2 < nbh, b + 2, 0)
                nxt1 = jnp.where(b + 3 < nbh, b + 3, 1)
                pltpu.make_async_copy(y4.at[pl.ds(0, BE)], rb0, gs0).wait()
                pltpu.async_copy(rb0, acc.at[dstv.at[b]], ss0, add=True)
                pltpu.make_async_copy(y4.at[pl.ds(0, BE)], rb1, gs1).wait()
                pltpu.async_copy(rb1, acc.at[dstv.at[b + 1]], ss1, add=True)
                pltpu.make_async_copy(rb0, acc.at[pl.ds(0, BE)], ss0).wait()
                pltpu.async_copy(y4.at[srcv.at[nxt0]], rb0, gs0)
                pltpu.make_async_copy(rb1, acc.at[pl.ds(0, BE)], ss1).wait()
                pltpu.async_copy(y4.at[srcv.at[nxt1]], rb1, gs1)
                return carry

            lax.fori_loop(0, nbh // 2, batch2, 0)
            pltpu.make_async_copy(y4.at[pl.ds(0, BE)], rb0, gs0).wait()
            pltpu.make_async_copy(y4.at[pl.ds(0, BE)], rb1, gs1).wait()
        plsc.subcore_barrier()
        pltpu.sync_copy(acc.at[pl.ds(r0, ROWS_PT)],
                        out.at[c, pl.ds(r0, ROWS_PT)])

    return segsum


_segsum80 = _make_segsum(80)
_segsum40 = _make_segsum(40)


def _hin_dots(y_ref, agg_ref, w1t_ref, cols, with_y):
    acc = None
    for k, col in enumerate(cols):
        hin = agg_ref[k]
        if with_y:
            hin = hin + (1.0 + EPS) * y_ref[:, col * W:(col + 1) * W]
        p = jnp.dot(hin, w1t_ref[col * W:(col + 1) * W, :],
                    preferred_element_type=jnp.float32)
        acc = p if acc is None else acc + p
    return acc


def _make_partial(fin, cols, with_y):
    def body(y_ref, agg_ref, w1t_ref, h1pre_ref):
        h1pre_ref[...] = _hin_dots(y_ref, agg_ref, w1t_ref, cols, with_y)

    return pl.pallas_call(
        body,
        grid=(GRID,),
        in_specs=[
            pl.BlockSpec((BN, fin), lambda i: (i, 0)),
            pl.BlockSpec((2, BN, W), lambda i: (0, i, 0)),
            pl.BlockSpec((fin, DIM1), lambda i: (0, 0)),
        ],
        out_specs=pl.BlockSpec((BN, DIM1), lambda i: (i, 0)),
        out_shape=jax.ShapeDtypeStruct((NPAD, DIM1), jnp.float32),
    )


def _make_finish(fin, cols, with_y):
    def body(y_ref, agg_ref, h1pre_ref, w1t_ref, b1_ref, w2t_ref, b2_ref,
             yn_ref, cs_ref):
        i = pl.program_id(0)
        h1pre = h1pre_ref[...] + _hin_dots(y_ref, agg_ref, w1t_ref, cols,
                                           with_y)
        h1 = jnp.tanh(h1pre + b1_ref[...])
        h2 = jnp.tanh(jnp.dot(h1, w2t_ref[...],
                              preferred_element_type=jnp.float32) + b2_ref[...])
        y = jnp.tanh(h2)
        yn_ref[...] = y
        rows = i * BN + lax.broadcasted_iota(jnp.int32, (BN, 1), 0)
        ym = jnp.where(rows < N, y, 0.0)

        @pl.when(i == 0)
        def _():
            cs_ref[...] = jnp.zeros_like(cs_ref)

        cs_ref[...] += jnp.sum(ym, axis=0, keepdims=True)

    return pl.pallas_call(
        body,
        grid=(GRID,),
        in_specs=[
            pl.BlockSpec((BN, fin), lambda i: (i, 0)),
            pl.BlockSpec((2, BN, W), lambda i: (0, i, 0)),
            pl.BlockSpec((BN, DIM1), lambda i: (i, 0)),
            pl.BlockSpec((fin, DIM1), lambda i: (0, 0)),
            pl.BlockSpec((1, DIM1), lambda i: (0, 0)),
            pl.BlockSpec((DIM1, DIM1), lambda i: (0, 0)),
            pl.BlockSpec((1, DIM1), lambda i: (0, 0)),
        ],
        out_specs=[
            pl.BlockSpec((BN, DIM1), lambda i: (i, 0)),
            pl.BlockSpec((1, DIM1), lambda i: (0, 0)),
        ],
        out_shape=[
            jax.ShapeDtypeStruct((NPAD, DIM1), jnp.float32),
            jax.ShapeDtypeStruct((1, DIM1), jnp.float32),
        ],
    )


_partial0 = _make_partial(256, (0, 1), True)
_finish0 = _make_finish(256, (0, 1), False)
_partial12 = _make_partial(512, (0, 2), True)
_finish12 = _make_finish(512, (1, 3), True)


def _head_body(cols_ref, wc1t_ref, bc1_ref, wc2t_ref, bc2_ref, out_ref):
    acc = None
    for j in range(R):
        p = jnp.dot(cols_ref[j:j + 1, :], wc1t_ref[j * DIM1:(j + 1) * DIM1, :],
                    preferred_element_type=jnp.float32)
        acc = p if acc is None else acc + p
    hidden = jnp.tanh(acc + bc1_ref[...])
    out_ref[...] = jnp.dot(hidden, wc2t_ref[...],
                           preferred_element_type=jnp.float32) + bc2_ref[...]


_head = pl.pallas_call(
    _head_body,
    out_shape=jax.ShapeDtypeStruct((1, NCLS), jnp.float32),
)


def kernel(x, edge_index, w1_0, b1_0, w2_0, b2_0, w1_1, b1_1, w2_1, b2_1,
           w1_2, b1_2, w2_2, b2_2, wc1, bc1, wc2, bc2):
    xp = jnp.pad(x, ((0, NPAD - N), (0, 0)))
    src = edge_index[0]
    dst = edge_index[1]
    pad_ids = N + (jnp.arange(PE - E, dtype=jnp.int32) % 16)
    srcp = jnp.concatenate([src, pad_ids])
    dstp = jnp.concatenate([dst, pad_ids])
    two = jnp.arange(2, dtype=jnp.int32)

    s2 = (srcp.reshape(1, NSUB, NB, BE) * 2
          + two[:, None, None, None])
    d2 = jnp.broadcast_to(dstp.reshape(1, NSUB, NB, BE), (2, NSUB, NB, BE))
    srcA0, srcB0 = s2[:, :, :40], s2[:, :, 40:]
    dstA0, dstB0 = d2[:, :, :40], d2[:, :, 40:]
    s4 = srcp.reshape(1, NSUB, NB, BE) * 4
    srcA1 = s4 + (two * 2)[:, None, None, None]
    srcB1 = s4 + (two * 2 + 1)[:, None, None, None]
    zeros = jnp.zeros((NPAD, W), jnp.float32)

    layers = [(w1_0, b1_0, w2_0, b2_0), (w1_1, b1_1, w2_1, b2_1),
              (w1_2, b1_2, w2_2, b2_2)]
    Y = xp
    cols = []
    for j, (w1, b1, w2, b2) in enumerate(layers):
        w1t = w1.T
        if j == 0:
            y4 = Y.reshape(NPAD * 2, W)
            aggA = _segsum40(y4, srcA0, dstA0, zeros)
            h1preA = _partial0(Y, aggA, w1t)
            aggB = _segsum40(y4, srcB0, dstB0, zeros)
            Y, cs = _finish0(Y, aggB, h1preA, w1t, b1.reshape(1, DIM1),
                             w2.T, b2.reshape(1, DIM1))
        else:
            y4 = Y.reshape(NPAD * 4, W)
            aggA = _segsum80(y4, srcA1, d2, zeros)
            h1preA = _partial12(Y, aggA, w1t)
            aggB = _segsum80(y4, srcB1, d2, zeros)
            Y, cs = _finish12(Y, aggB, h1preA, w1t, b1.reshape(1, DIM1),
                              w2.T, b2.reshape(1, DIM1))
        cols.append(cs)

    colsmat = jnp.concatenate(cols, axis=0)
    wc1p = wc1.reshape(DIM1, DIM1, R).transpose(0, 2, 1).reshape(DIM1, R * DIM1)
    return _head(colsmat, wc1p.T, bc1.reshape(1, DIM1), wc2.T,
                 bc2.reshape(1, NCLS))


# speedup vs baseline: 1.1756x; 1.1756x over previous
import functools

import jax
import jax.numpy as jnp
from jax import lax
from jax.experimental import pallas as pl
from jax.experimental.pallas import tpu as pltpu
from jax.experimental.pallas import tpu_sc as plsc

N = 10000
E = 160000
D = 256
DIM1 = 512
R = 3
NCLS = 10
EPS = 0.1

NPAD = 10240
W = 128
NSUB = 16
NCORE = 2
BE = 64
NB = 160
PE = NSUB * NB * BE
ROWS_PT = NPAD // NSUB
NBUF = 4
NSTG = 40
BN = 1024
GRID = NPAD // BN


def _make_segsum(nchunk):
    cpc = nchunk // NCORE
    mesh = plsc.VectorSubcoreMesh(core_axis_name="c", subcore_axis_name="s")

    @functools.partial(
        pl.kernel,
        mesh=mesh,
        out_type=jax.ShapeDtypeStruct((nchunk, NPAD, W), jnp.float32),
        scratch_types=(
            [pltpu.VMEM((NSTG, BE), jnp.int32),
             pltpu.VMEM((NSTG, BE), jnp.int32)]
            + [pltpu.VMEM((BE, W), jnp.float32) for _ in range(NBUF)]
            + [pltpu.VMEM_SHARED((NPAD, W), jnp.float32)]
            + [pltpu.SemaphoreType.DMA for _ in range(2 * NBUF)]
        ),
    )
    def segsum(ytab, srci, dsti, zeros, agg, *scr):
        srcv, dstv = scr[0], scr[1]
        bufs = scr[2:2 + NBUF]
        acc = scr[2 + NBUF]
        gsems = scr[3 + NBUF:3 + 2 * NBUF]
        ssems = scr[3 + 2 * NBUF:3 + 3 * NBUF]
        c = lax.axis_index("c")
        s = lax.axis_index("s")
        r0 = s * ROWS_PT
        pltpu.sync_copy(zeros.at[pl.ds(r0, ROWS_PT)], acc.at[pl.ds(r0, ROWS_PT)])
        plsc.subcore_barrier()
        for ci in range(cpc):
            ch = c * cpc + ci
            for stg in range(NB // NSTG):
                h0 = stg * NSTG
                pltpu.sync_copy(srci.at[ch, s, pl.ds(h0, NSTG)], srcv)
                pltpu.sync_copy(dsti.at[s, pl.ds(h0, NSTG)], dstv)
                for u in range(NBUF):
                    pltpu.async_copy(ytab.at[srcv.at[u]], bufs[u], gsems[u])

                def wave(k, carry):
                    b = k * NBUF
                    for u in range(NBUF):
                        pltpu.make_async_copy(ytab.at[pl.ds(0, BE)], bufs[u],
                                              gsems[u]).wait()
                        pltpu.async_copy(bufs[u], acc.at[dstv.at[b + u]],
                                         ssems[u], add=True)
                    for u in range(NBUF):
                        nxt = jnp.where(b + NBUF + u < NSTG, b + NBUF + u, u)
                        pltpu.make_async_copy(bufs[u], acc.at[pl.ds(0, BE)],
                                              ssems[u]).wait()
                        pltpu.async_copy(ytab.at[srcv.at[nxt]], bufs[u],
                                        gsems[u])
                    return carry

                lax.fori_loop(0, NSTG // NBUF, wave, 0)
                for u in range(NBUF):
                    pltpu.make_async_copy(ytab.at[pl.ds(0, BE)], bufs[u],
                                          gsems[u]).wait()
            plsc.subcore_barrier()
            pltpu.sync_copy(acc.at[pl.ds(r0, ROWS_PT)],
                            agg.at[ch, pl.ds(r0, ROWS_PT)])
            if ci + 1 < cpc:
                pltpu.sync_copy(zeros.at[pl.ds(r0, ROWS_PT)],
                                acc.at[pl.ds(r0, ROWS_PT)])
                plsc.subcore_barrier()

    return segsum


_segsum2 = _make_segsum(2)
_segsum4 = _make_segsum(4)


def _mlp_body(nchunk):
    def body(y_ref, agg_ref, w1t_ref, b1_ref, w2t_ref, b2_ref, yn_ref, cs_ref):
        i = pl.program_id(0)
        h1pre = None
        for ci in range(nchunk):
            hin = (1.0 + EPS) * y_ref[:, ci * W:(ci + 1) * W] + agg_ref[ci]
            p = jnp.dot(hin, w1t_ref[ci * W:(ci + 1) * W, :],
                        preferred_element_type=jnp.float32)
            h1pre = p if h1pre is None else h1pre + p
        h1 = jnp.tanh(h1pre + b1_ref[...])
        h2 = jnp.tanh(jnp.dot(h1, w2t_ref[...],
                              preferred_element_type=jnp.float32) + b2_ref[...])
        y = jnp.tanh(h2)
        yn_ref[...] = y
        rows = i * BN + lax.broadcasted_iota(jnp.int32, (BN, 1), 0)
        ym = jnp.where(rows < N, y, 0.0)

        @pl.when(i == 0)
        def _():
            cs_ref[...] = jnp.zeros_like(cs_ref)

        cs_ref[...] += jnp.sum(ym, axis=0, keepdims=True)

    return body


def _make_mlp(nchunk):
    fin = nchunk * W
    return pl.pallas_call(
        _mlp_body(nchunk),
        grid=(GRID,),
        in_specs=[
            pl.BlockSpec((BN, fin), lambda i: (i, 0)),
            pl.BlockSpec((nchunk, BN, W), lambda i: (0, i, 0)),
            pl.BlockSpec((fin, DIM1), lambda i: (0, 0)),
            pl.BlockSpec((1, DIM1), lambda i: (0, 0)),
            pl.BlockSpec((DIM1, DIM1), lambda i: (0, 0)),
            pl.BlockSpec((1, DIM1), lambda i: (0, 0)),
        ],
        out_specs=[
            pl.BlockSpec((BN, DIM1), lambda i: (i, 0)),
            pl.BlockSpec((1, DIM1), lambda i: (0, 0)),
        ],
        out_shape=[
            jax.ShapeDtypeStruct((NPAD, DIM1), jnp.float32),
            jax.ShapeDtypeStruct((1, DIM1), jnp.float32),
        ],
    )


_mlp2 = _make_mlp(2)
_mlp4 = _make_mlp(4)


def _head_body(cols_ref, wc1t_ref, bc1_ref, wc2t_ref, bc2_ref, out_ref):
    acc = None
    for j in range(R):
        p = jnp.dot(cols_ref[j:j + 1, :], wc1t_ref[j * DIM1:(j + 1) * DIM1, :],
                    preferred_element_type=jnp.float32)
        acc = p if acc is None else acc + p
    hidden = jnp.tanh(acc + bc1_ref[...])
    out_ref[...] = jnp.dot(hidden, wc2t_ref[...],
                           preferred_element_type=jnp.float32) + bc2_ref[...]


_head = pl.pallas_call(
    _head_body,
    out_shape=jax.ShapeDtypeStruct((1, NCLS), jnp.float32),
)


def kernel(x, edge_index, w1_0, b1_0, w2_0, b2_0, w1_1, b1_1, w2_1, b2_1,
           w1_2, b1_2, w2_2, b2_2, wc1, bc1, wc2, bc2):
    xp = jnp.pad(x, ((0, NPAD - N), (0, 0)))
    src = edge_index[0]
    dst = edge_index[1]
    pad_ids = N + (jnp.arange(PE - E, dtype=jnp.int32) % 16)
    srcp = jnp.concatenate([src, pad_ids])
    dstp = jnp.concatenate([dst, pad_ids])
    dst3 = dstp.reshape(NSUB, NB, BE)
    idx2 = (srcp[None, :] * 2 +
            jnp.arange(2, dtype=jnp.int32)[:, None]).reshape(2, NSUB, NB, BE)
    idx4 = (srcp[None, :] * 4 +
            jnp.arange(4, dtype=jnp.int32)[:, None]).reshape(4, NSUB, NB, BE)
    zeros = jnp.zeros((NPAD, W), jnp.float32)

    layers = [(w1_0, b1_0, w2_0, b2_0), (w1_1, b1_1, w2_1, b2_1),
              (w1_2, b1_2, w2_2, b2_2)]
    Y = xp
    cols = []
    for j, (w1, b1, w2, b2) in enumerate(layers):
        nchunk = (D if j == 0 else DIM1) // W
        ytab = Y.reshape(NPAD * nchunk, W)
        if nchunk == 2:
            agg = _segsum2(ytab, idx2, dst3, zeros)
            Y, cs = _mlp2(Y, agg, w1.T, b1.reshape(1, DIM1), w2.T,
                          b2.reshape(1, DIM1))
        else:
            agg = _segsum4(ytab, idx4, dst3, zeros)
            Y, cs = _mlp4(Y, agg, w1.T, b1.reshape(1, DIM1), w2.T,
                          b2.reshape(1, DIM1))
        cols.append(cs)

    colsmat = jnp.concatenate(cols, axis=0)
    wc1p = wc1.reshape(DIM1, DIM1, R).transpose(0, 2, 1).reshape(DIM1, R * DIM1)
    return _head(colsmat, wc1p.T, bc1.reshape(1, DIM1), wc2.T,
                 bc2.reshape(1, NCLS))


# speedup vs baseline: 1.2010x; 1.0216x over previous
import functools

import jax
import jax.numpy as jnp
from jax import lax
from jax.experimental import pallas as pl
from jax.experimental.pallas import tpu as pltpu
from jax.experimental.pallas import tpu_sc as plsc

N = 10000
E = 160000
D = 256
DIM1 = 512
R = 3
NCLS = 10
EPS = 0.1

NPAD = 10240
W = 128
NSUB = 16
NCORE = 2
BE = 64
NB = 160
PE = NSUB * NB * BE
ROWS_PT = NPAD // NSUB
NBUF = 4
NSTG = 32
BN = 1024
GRID = NPAD // BN


def _make_segsum(nchunk):
    cpc = nchunk // NCORE
    mesh = plsc.VectorSubcoreMesh(core_axis_name="c", subcore_axis_name="s")

    @functools.partial(
        pl.kernel,
        mesh=mesh,
        out_type=jax.ShapeDtypeStruct((nchunk, NPAD, W), jnp.float32),
        scratch_types=(
            [pltpu.VMEM((NSTG, BE), jnp.int32),
             pltpu.VMEM((NSTG, BE), jnp.int32),
             pltpu.VMEM((NSTG, BE), jnp.int32),
             pltpu.VMEM((NSTG, BE), jnp.int32)]
            + [pltpu.VMEM((BE, W), jnp.float32) for _ in range(NBUF)]
            + [pltpu.VMEM_SHARED((NPAD, W), jnp.float32)]
            + [pltpu.SemaphoreType.DMA for _ in range(2 * NBUF + 2)]
        ),
    )
    def segsum(ytab, srci, dsti, zeros, agg, *scr):
        srcvs = scr[0:2]
        dstvs = scr[2:4]
        bufs = scr[4:4 + NBUF]
        acc = scr[4 + NBUF]
        gsems = scr[5 + NBUF:5 + 2 * NBUF]
        ssems = scr[5 + 2 * NBUF:5 + 3 * NBUF]
        isems = scr[5 + 3 * NBUF:7 + 3 * NBUF]
        c = lax.axis_index("c")
        s = lax.axis_index("s")
        r0 = s * ROWS_PT
        nstage = NB // NSTG
        pltpu.async_copy(srci.at[c * cpc, s, pl.ds(0, NSTG)], srcvs[0],
                         isems[0])
        pltpu.async_copy(dsti.at[s, pl.ds(0, NSTG)], dstvs[0], isems[1])
        pltpu.sync_copy(zeros.at[pl.ds(r0, ROWS_PT)], acc.at[pl.ds(r0, ROWS_PT)])
        plsc.subcore_barrier()
        for ci in range(cpc):
            ch = c * cpc + ci
            for stg in range(nstage):
                g = ci * nstage + stg
                pb = g % 2
                srcv, dstv = srcvs[pb], dstvs[pb]
                pltpu.make_async_copy(srci.at[ch, s, pl.ds(0, NSTG)],
                                      srcv, isems[0]).wait()
                pltpu.make_async_copy(dsti.at[s, pl.ds(0, NSTG)],
                                      dstv, isems[1]).wait()
                if g + 1 < cpc * nstage:
                    nch = c * cpc + (g + 1) // nstage
                    nh0 = ((g + 1) % nstage) * NSTG
                    pltpu.async_copy(srci.at[nch, s, pl.ds(nh0, NSTG)],
                                     srcvs[1 - pb], isems[0])
                    pltpu.async_copy(dsti.at[s, pl.ds(nh0, NSTG)],
                                     dstvs[1 - pb], isems[1])
                for u in range(NBUF):
                    pltpu.async_copy(ytab.at[srcv.at[u]], bufs[u], gsems[u])

                def wave(k, carry):
                    b = k * NBUF
                    for u in range(NBUF):
                        pltpu.make_async_copy(ytab.at[pl.ds(0, BE)], bufs[u],
                                              gsems[u]).wait()
                        pltpu.async_copy(bufs[u], acc.at[dstv.at[b + u]],
                                         ssems[u], add=True)
                    for u in range(NBUF):
                        nxt = jnp.where(b + NBUF + u < NSTG, b + NBUF + u, u)
                        pltpu.make_async_copy(bufs[u], acc.at[pl.ds(0, BE)],
                                              ssems[u]).wait()
                        pltpu.async_copy(ytab.at[srcv.at[nxt]], bufs[u],
                                        gsems[u])
                    return carry

                lax.fori_loop(0, NSTG // NBUF, wave, 0)
                for u in range(NBUF):
                    pltpu.make_async_copy(ytab.at[pl.ds(0, BE)], bufs[u],
                                          gsems[u]).wait()
            plsc.subcore_barrier()
            pltpu.sync_copy(acc.at[pl.ds(r0, ROWS_PT)],
                            agg.at[ch, pl.ds(r0, ROWS_PT)])
            if ci + 1 < cpc:
                pltpu.sync_copy(zeros.at[pl.ds(r0, ROWS_PT)],
                                acc.at[pl.ds(r0, ROWS_PT)])
                plsc.subcore_barrier()

    return segsum


_segsum2 = _make_segsum(2)
_segsum4 = _make_segsum(4)


def _mlp_body(nchunk):
    def body(y_ref, agg_ref, w1t_ref, b1_ref, w2t_ref, b2_ref, yn_ref, cs_ref):
        i = pl.program_id(0)
        h1pre = None
        for ci in range(nchunk):
            hin = (1.0 + EPS) * y_ref[:, ci * W:(ci + 1) * W] + agg_ref[ci]
            p = jnp.dot(hin, w1t_ref[ci * W:(ci + 1) * W, :],
                        preferred_element_type=jnp.float32)
            h1pre = p if h1pre is None else h1pre + p
        h1 = jnp.tanh(h1pre + b1_ref[...])
        h2 = jnp.tanh(jnp.dot(h1, w2t_ref[...],
                              preferred_element_type=jnp.float32) + b2_ref[...])
        y = jnp.tanh(h2)
        yn_ref[...] = y
        rows = i * BN + lax.broadcasted_iota(jnp.int32, (BN, 1), 0)
        ym = jnp.where(rows < N, y, 0.0)

        @pl.when(i == 0)
        def _():
            cs_ref[...] = jnp.zeros_like(cs_ref)

        cs_ref[...] += jnp.sum(ym, axis=0, keepdims=True)

    return body


def _make_mlp(nchunk):
    fin = nchunk * W
    return pl.pallas_call(
        _mlp_body(nchunk),
        grid=(GRID,),
        in_specs=[
            pl.BlockSpec((BN, fin), lambda i: (i, 0)),
            pl.BlockSpec((nchunk, BN, W), lambda i: (0, i, 0)),
            pl.BlockSpec((fin, DIM1), lambda i: (0, 0)),
            pl.BlockSpec((1, DIM1), lambda i: (0, 0)),
            pl.BlockSpec((DIM1, DIM1), lambda i: (0, 0)),
            pl.BlockSpec((1, DIM1), lambda i: (0, 0)),
        ],
        out_specs=[
            pl.BlockSpec((BN, DIM1), lambda i: (i, 0)),
            pl.BlockSpec((1, DIM1), lambda i: (0, 0)),
        ],
        out_shape=[
            jax.ShapeDtypeStruct((NPAD, DIM1), jnp.float32),
            jax.ShapeDtypeStruct((1, DIM1), jnp.float32),
        ],
    )


_mlp2 = _make_mlp(2)
_mlp4 = _make_mlp(4)


def _head_body(cols_ref, wc1t_ref, bc1_ref, wc2t_ref, bc2_ref, out_ref):
    acc = None
    for j in range(R):
        p = jnp.dot(cols_ref[j:j + 1, :], wc1t_ref[j * DIM1:(j + 1) * DIM1, :],
                    preferred_element_type=jnp.float32)
        acc = p if acc is None else acc + p
    hidden = jnp.tanh(acc + bc1_ref[...])
    out_ref[...] = jnp.dot(hidden, wc2t_ref[...],
                           preferred_element_type=jnp.float32) + bc2_ref[...]


_head = pl.pallas_call(
    _head_body,
    out_shape=jax.ShapeDtypeStruct((1, NCLS), jnp.float32),
)


def kernel(x, edge_index, w1_0, b1_0, w2_0, b2_0, w1_1, b1_1, w2_1, b2_1,
           w1_2, b1_2, w2_2, b2_2, wc1, bc1, wc2, bc2):
    xp = jnp.pad(x, ((0, NPAD - N), (0, 0)))
    src = edge_index[0]
    dst = edge_index[1]
    pad_ids = N + (jnp.arange(PE - E, dtype=jnp.int32) % 16)
    srcp = jnp.concatenate([src, pad_ids])
    dstp = jnp.concatenate([dst, pad_ids])
    dst3 = dstp.reshape(NSUB, NB, BE)
    idx2 = (srcp[None, :] * 2 +
            jnp.arange(2, dtype=jnp.int32)[:, None]).reshape(2, NSUB, NB, BE)
    idx4 = (srcp[None, :] * 4 +
            jnp.arange(4, dtype=jnp.int32)[:, None]).reshape(4, NSUB, NB, BE)
    zeros = jnp.zeros((NPAD, W), jnp.float32)

    layers = [(w1_0, b1_0, w2_0, b2_0), (w1_1, b1_1, w2_1, b2_1),
              (w1_2, b1_2, w2_2, b2_2)]
    Y = xp
    cols = []
    for j, (w1, b1, w2, b2) in enumerate(layers):
        nchunk = (D if j == 0 else DIM1) // W
        ytab = Y.reshape(NPAD * nchunk, W)
        if nchunk == 2:
            agg = _segsum2(ytab, idx2, dst3, zeros)
            Y, cs = _mlp2(Y, agg, w1.T, b1.reshape(1, DIM1), w2.T,
                          b2.reshape(1, DIM1))
        else:
            agg = _segsum4(ytab, idx4, dst3, zeros)
            Y, cs = _mlp4(Y, agg, w1.T, b1.reshape(1, DIM1), w2.T,
                          b2.reshape(1, DIM1))
        cols.append(cs)

    colsmat = jnp.concatenate(cols, axis=0)
    wc1p = wc1.reshape(DIM1, DIM1, R).transpose(0, 2, 1).reshape(DIM1, R * DIM1)
    return _head(colsmat, wc1p.T, bc1.reshape(1, DIM1), wc2.T,
                 bc2.reshape(1, NCLS))
